# baseline (speedup 1.0000x reference)
import jax
import jax.numpy as jnp
from jax import lax
from jax.experimental import pallas as pl
from jax.experimental.pallas import tpu as pltpu
from jax.experimental.pallas import tpu_sc as plsc

N = 10000
E = 160000
ATOM_DIM, BOND_DIM, EMB, LAYERS, OUT = 150, 12, 300, 5, 118

NQ = 4
QW = EMB // NQ
QP = 80
NC, NS, LANES = 2, 16, 16
CH = 112
EPT = 10080
NCHUNK = EPT // CH
EPAD = EPT * NS
NPAD = 10112
ROWS_PER_TILE = NPAD // NS
TRASH = N + 1
NG = 3
NSB = 3
UNROLL = 3


def _sc_layer_body(hs_hbm, e_hbm, src_hbm, dst_hbm, out_hbm,
                   dstv, isrc, hrows, sbuf, aggr, gsem, esem, ssem, isem):
    c = lax.axis_index("c")
    s = lax.axis_index("s")
    stripe = pl.ds(s * ROWS_PER_TILE, ROWS_PER_TILE)
    ebase = s * EPT

    pltpu.sync_copy(dst_hbm.at[s], dstv)

    def idx_cp(k, b, q):
        return pltpu.make_async_copy(
            src_hbm.at[pl.ds(((q * NS + s) * NCHUNK + k) * CH, CH)],
            isrc[b], isem[b])

    def gather_cp(b):
        return pltpu.make_async_copy(hs_hbm.at[isrc[b]], hrows[b], gsem[b])

    def eload_cp(k, b, q):
        return pltpu.make_async_copy(
            e_hbm.at[q, pl.ds(ebase + k * CH, CH)], sbuf[b], esem[b])

    def scatter_cp(k, b):
        return pltpu.make_async_copy(sbuf[b], aggr.at[dstv.at[k]], ssem[b])

    for p in range(2):
        q = c * 2 + p
        @plsc.parallel_loop(0, CH, unroll=4)
        def _zero(i):
            for j in range(QP // LANES):
                sbuf[0][i, pl.ds(j * LANES, LANES)] = jnp.zeros(
                    (LANES,), jnp.float32)

        for m in range(ROWS_PER_TILE // CH):
            pltpu.sync_copy(
                sbuf[0], aggr.at[pl.ds(s * ROWS_PER_TILE + m * CH, CH)])
        _tail = ROWS_PER_TILE % CH
        if _tail:
            _toff = s * ROWS_PER_TILE + (ROWS_PER_TILE // CH) * CH
            pltpu.sync_copy(sbuf[0].at[pl.ds(0, _tail)],
                            aggr.at[pl.ds(_toff, _tail)])
        plsc.subcore_barrier()

        for b in range(NG):
            idx_cp(b, b, q).start()
        for b in range(NG - 1):
            idx_cp(b, b, q).wait()
            gather_cp(b).start()
        eload_cp(0, 0, q).start()

        def step(k, b5, b3):
            @pl.when(k + 2 < NCHUNK)
            def _():
                idx_cp(k + 2, (b5 + 2) % NG, q).wait()
                gather_cp((b5 + 2) % NG).start()

            @pl.when(k + 1 < NCHUNK)
            def _():
                @pl.when(k >= 2)
                def _():
                    scatter_cp(k - 2, (b3 + 1) % NSB).wait()
                eload_cp(k + 1, (b3 + 1) % NSB, q).start()

            gather_cp(b5).wait()
            eload_cp(k, b3, q).wait()

            @pl.when(k + 3 < NCHUNK)
            def _():
                idx_cp(k + 3, b5, q).start()

            @plsc.parallel_loop(0, CH, unroll=4)
            def _compute(i):
                for j in range(QP // LANES):
                    sl = pl.ds(j * LANES, LANES)
                    sbuf[b3][i, sl] = jnp.maximum(
                        sbuf[b3][i, sl] + hrows[b5][i, sl], 0.0)

            scatter_cp(k, b3).start(add=True)

        def block(g, carry):
            for u in range(UNROLL):
                k0 = UNROLL * g + u
                step(k0, u % NG, u % NSB)
            return carry

        lax.fori_loop(0, NCHUNK // UNROLL, block, 0)
        for k in (NCHUNK - 3, NCHUNK - 2, NCHUNK - 1):
            scatter_cp(k, k % NSB).wait()
        plsc.subcore_barrier()
        pltpu.sync_copy(aggr.at[stripe], out_hbm.at[q, stripe])


_sc_layer = pl.kernel(
    _sc_layer_body,
    out_type=jax.ShapeDtypeStruct((NQ, NPAD, QP), jnp.float32),
    mesh=plsc.VectorSubcoreMesh(core_axis_name="c", subcore_axis_name="s",
                                num_cores=NC, num_subcores=NS),
    scratch_types=[
        pltpu.VMEM((NCHUNK, CH), jnp.int32),
        [pltpu.VMEM((CH,), jnp.int32)] * NG,
        [pltpu.VMEM((CH, QP), jnp.float32)] * NG,
        [pltpu.VMEM((CH, QP), jnp.float32)] * NSB,
        pltpu.VMEM_SHARED((NPAD, QP), jnp.float32),
        [pltpu.SemaphoreType.DMA] * NG,
        [pltpu.SemaphoreType.DMA] * NSB,
        [pltpu.SemaphoreType.DMA] * NSB,
        [pltpu.SemaphoreType.DMA] * NG,
    ],
    compiler_params=pltpu.CompilerParams(use_tc_tiling_on_sc=False),
)


def _split_pack(r, bm):
    z = jnp.zeros((bm, QP - QW), jnp.float32)
    return jnp.stack(
        [jnp.concatenate([r[:, q * QW:(q + 1) * QW], z], axis=1)
         for q in range(NQ)], axis=0)


def _unsplit(hs):
    return jnp.concatenate([hs[q, :, :QW] for q in range(NQ)], axis=1)


def _embed_body(x_ref, w_ref, b_ref, out_ref):
    r = jnp.dot(x_ref[...], w_ref[...],
                preferred_element_type=jnp.float32) + b_ref[0]
    out_ref[...] = _split_pack(r, r.shape[0])


def _layer_update_body(hs_ref, ag_ref, w_ref, b_ref, out_ref):
    h = _unsplit(hs_ref[...])
    a = _unsplit(ag_ref[...])
    h2 = jnp.dot(h + a, w_ref[...], preferred_element_type=jnp.float32)
    hn = jnp.maximum(h2 + b_ref[0], 0.0) + h
    out_ref[...] = _split_pack(hn, hn.shape[0])


def _mlp_body(hs_ref, w_ref, b_ref, out_ref):
    h = _unsplit(hs_ref[...])
    out_ref[...] = jnp.dot(h, w_ref[...],
                           preferred_element_type=jnp.float32) + b_ref[0]


_BM = 2000
_BE = 1920

_embed_atoms = pl.pallas_call(
    _embed_body,
    grid=(N // _BM,),
    in_specs=[
        pl.BlockSpec((_BM, ATOM_DIM), lambda i: (i, 0)),
        pl.BlockSpec((ATOM_DIM, EMB), lambda i: (0, 0)),
        pl.BlockSpec((1, EMB), lambda i: (0, 0)),
    ],
    out_specs=pl.BlockSpec((NQ, _BM, QP), lambda i: (0, i, 0)),
    out_shape=jax.ShapeDtypeStruct((NQ, N, QP), jnp.float32),
)

_embed_bonds = pl.pallas_call(
    _embed_body,
    grid=(EPAD // _BE,),
    in_specs=[
        pl.BlockSpec((_BE, BOND_DIM), lambda i: (i, 0)),
        pl.BlockSpec((BOND_DIM, EMB), lambda i: (0, 0)),
        pl.BlockSpec((1, EMB), lambda i: (0, 0)),
    ],
    out_specs=pl.BlockSpec((NQ, _BE, QP), lambda i: (0, i, 0)),
    out_shape=jax.ShapeDtypeStruct((NQ, EPAD, QP), jnp.float32),
)

_layer_update = pl.pallas_call(
    _layer_update_body,
    grid=(N // _BM,),
    in_specs=[
        pl.BlockSpec((NQ, _BM, QP), lambda i: (0, i, 0)),
        pl.BlockSpec((NQ, _BM, QP), lambda i: (0, i, 0)),
        pl.BlockSpec((EMB, EMB), lambda i: (0, 0)),
        pl.BlockSpec((1, EMB), lambda i: (0, 0)),
    ],
    out_specs=pl.BlockSpec((NQ, _BM, QP), lambda i: (0, i, 0)),
    out_shape=jax.ShapeDtypeStruct((NQ, N, QP), jnp.float32),
)

_mlp_head = pl.pallas_call(
    _mlp_body,
    grid=(N // _BM,),
    in_specs=[
        pl.BlockSpec((NQ, _BM, QP), lambda i: (0, i, 0)),
        pl.BlockSpec((EMB, OUT), lambda i: (0, 0)),
        pl.BlockSpec((1, OUT), lambda i: (0, 0)),
    ],
    out_specs=pl.BlockSpec((_BM, OUT), lambda i: (i, 0)),
    out_shape=jax.ShapeDtypeStruct((N, OUT), jnp.float32),
)


def kernel(atom_feat, bond_feat, edge_index, W_atom, b_atom, W_bond, b_bond,
           Wg, bg, W_mlp, b_mlp):
    src = edge_index[0]
    dst = edge_index[1]
    src_pad = jnp.pad(src, (0, EPAD - E))
    dst_pad = jnp.pad(dst, (0, EPAD - E), constant_values=TRASH)
    src4 = (src_pad[None, :] +
            (N * jnp.arange(NQ, dtype=jnp.int32))[:, None]).reshape(-1)
    dst3 = dst_pad.reshape(NS, NCHUNK, CH)

    hs = _embed_atoms(atom_feat, W_atom, b_atom.reshape(1, EMB))
    e = _embed_bonds(bond_feat, W_bond, b_bond.reshape(1, EMB))

    for i in range(LAYERS):
        aggr = _sc_layer(hs.reshape(NQ * N, QP), e, src4, dst3)
        hs = _layer_update(hs, aggr, Wg[i], bg[i].reshape(1, EMB))

    return _mlp_head(hs, W_mlp, b_mlp.reshape(1, OUT))


# speedup vs baseline: 1.0113x; 1.0113x over previous
import jax
import jax.numpy as jnp
from jax import lax
from jax.experimental import pallas as pl
from jax.experimental.pallas import tpu as pltpu
from jax.experimental.pallas import tpu_sc as plsc

N = 10000
E = 160000
ATOM_DIM, BOND_DIM, EMB, LAYERS, OUT = 150, 12, 300, 5, 118

NQ = 4
QW = EMB // NQ
QP = 80
NC, NS, LANES = 2, 16, 16
CH = 112
EPT = 10080
NCHUNK = EPT // CH
EPAD = EPT * NS
NPAD = 10112
ROWS_PER_TILE = NPAD // NS
TRASH = N + 1
NG = 3
NSB = 3
UNROLL = 3


def _sc_layer_body(hs_hbm, e_hbm, src_hbm, dst_hbm, zeros_hbm, out_hbm,
                   dstv, isrc, hrows, sbuf, aggr, gsem, esem, ssem, isem):
    c = lax.axis_index("c")
    s = lax.axis_index("s")
    stripe = pl.ds(s * ROWS_PER_TILE, ROWS_PER_TILE)
    ebase = s * EPT

    pltpu.sync_copy(dst_hbm.at[s], dstv)

    def idx_cp(k, b, q):
        return pltpu.make_async_copy(
            src_hbm.at[pl.ds(((q * NS + s) * NCHUNK + k) * CH, CH)],
            isrc[b], isem[b])

    def gather_cp(b):
        return pltpu.make_async_copy(hs_hbm.at[isrc[b]], hrows[b], gsem[b])

    def eload_cp(k, b, q):
        return pltpu.make_async_copy(
            e_hbm.at[q, pl.ds(ebase + k * CH, CH)], sbuf[b], esem[b])

    def scatter_cp(k, b):
        return pltpu.make_async_copy(sbuf[b], aggr.at[dstv.at[k]], ssem[b])

    for p in range(2):
        q = c * 2 + p
        pltpu.sync_copy(zeros_hbm.at[stripe], aggr.at[stripe])
        plsc.subcore_barrier()

        for b in range(NG):
            idx_cp(b, b, q).start()
        for b in range(NG - 1):
            idx_cp(b, b, q).wait()
            gather_cp(b).start()
        eload_cp(0, 0, q).start()

        def step(k, b5, b3):
            @pl.when(k + 2 < NCHUNK)
            def _():
                idx_cp(k + 2, (b5 + 2) % NG, q).wait()
                gather_cp((b5 + 2) % NG).start()

            @pl.when(k + 1 < NCHUNK)
            def _():
                @pl.when(k >= 2)
                def _():
                    scatter_cp(k - 2, (b3 + 1) % NSB).wait()
                eload_cp(k + 1, (b3 + 1) % NSB, q).start()

            gather_cp(b5).wait()
            eload_cp(k, b3, q).wait()

            @pl.when(k + 3 < NCHUNK)
            def _():
                idx_cp(k + 3, b5, q).start()

            @plsc.parallel_loop(0, CH, unroll=4)
            def _compute(i):
                for j in range(QP // LANES):
                    sl = pl.ds(j * LANES, LANES)
                    sbuf[b3][i, sl] = jnp.maximum(
                        sbuf[b3][i, sl] + hrows[b5][i, sl], 0.0)

            scatter_cp(k, b3).start(add=True)

        def block(g, carry):
            for u in range(UNROLL):
                k0 = UNROLL * g + u
                step(k0, u % NG, u % NSB)
            return carry

        lax.fori_loop(0, NCHUNK // UNROLL, block, 0)
        for k in (NCHUNK - 3, NCHUNK - 2, NCHUNK - 1):
            scatter_cp(k, k % NSB).wait()
        plsc.subcore_barrier()
        pltpu.sync_copy(aggr.at[stripe], out_hbm.at[q, stripe])


_sc_layer = pl.kernel(
    _sc_layer_body,
    out_type=jax.ShapeDtypeStruct((NQ, NPAD, QP), jnp.float32),
    mesh=plsc.VectorSubcoreMesh(core_axis_name="c", subcore_axis_name="s",
                                num_cores=NC, num_subcores=NS),
    scratch_types=[
        pltpu.VMEM((NCHUNK, CH), jnp.int32),
        [pltpu.VMEM((CH,), jnp.int32)] * NG,
        [pltpu.VMEM((CH, QP), jnp.float32)] * NG,
        [pltpu.VMEM((CH, QP), jnp.float32)] * NSB,
        pltpu.VMEM_SHARED((NPAD, QP), jnp.float32),
        [pltpu.SemaphoreType.DMA] * NG,
        [pltpu.SemaphoreType.DMA] * NSB,
        [pltpu.SemaphoreType.DMA] * NSB,
        [pltpu.SemaphoreType.DMA] * NG,
    ],
    compiler_params=pltpu.CompilerParams(use_tc_tiling_on_sc=False),
)


def _split_pack(r, bm):
    z = jnp.zeros((bm, QP - QW), jnp.float32)
    return jnp.stack(
        [jnp.concatenate([r[:, q * QW:(q + 1) * QW], z], axis=1)
         for q in range(NQ)], axis=0)


def _unsplit(hs):
    return jnp.concatenate([hs[q, :, :QW] for q in range(NQ)], axis=1)


def _embed_body(x_ref, w_ref, b_ref, out_ref):
    r = jnp.dot(x_ref[...], w_ref[...],
                preferred_element_type=jnp.float32) + b_ref[0]
    out_ref[...] = _split_pack(r, r.shape[0])


def _layer_update_body(hs_ref, ag_ref, w_ref, b_ref, out_ref):
    h = _unsplit(hs_ref[...])
    a = _unsplit(ag_ref[...])
    h2 = jnp.dot(h + a, w_ref[...], preferred_element_type=jnp.float32)
    hn = jnp.maximum(h2 + b_ref[0], 0.0) + h
    out_ref[...] = _split_pack(hn, hn.shape[0])


def _mlp_body(hs_ref, w_ref, b_ref, out_ref):
    h = _unsplit(hs_ref[...])
    out_ref[...] = jnp.dot(h, w_ref[...],
                           preferred_element_type=jnp.float32) + b_ref[0]


_BM = 2000
_BE = 1920

_embed_atoms = pl.pallas_call(
    _embed_body,
    grid=(N // _BM,),
    in_specs=[
        pl.BlockSpec((_BM, ATOM_DIM), lambda i: (i, 0)),
        pl.BlockSpec((ATOM_DIM, EMB), lambda i: (0, 0)),
        pl.BlockSpec((1, EMB), lambda i: (0, 0)),
    ],
    out_specs=pl.BlockSpec((NQ, _BM, QP), lambda i: (0, i, 0)),
    out_shape=jax.ShapeDtypeStruct((NQ, N, QP), jnp.float32),
)

_embed_bonds = pl.pallas_call(
    _embed_body,
    grid=(EPAD // _BE,),
    in_specs=[
        pl.BlockSpec((_BE, BOND_DIM), lambda i: (i, 0)),
        pl.BlockSpec((BOND_DIM, EMB), lambda i: (0, 0)),
        pl.BlockSpec((1, EMB), lambda i: (0, 0)),
    ],
    out_specs=pl.BlockSpec((NQ, _BE, QP), lambda i: (0, i, 0)),
    out_shape=jax.ShapeDtypeStruct((NQ, EPAD, QP), jnp.float32),
)

_layer_update = pl.pallas_call(
    _layer_update_body,
    grid=(N // _BM,),
    in_specs=[
        pl.BlockSpec((NQ, _BM, QP), lambda i: (0, i, 0)),
        pl.BlockSpec((NQ, _BM, QP), lambda i: (0, i, 0)),
        pl.BlockSpec((EMB, EMB), lambda i: (0, 0)),
        pl.BlockSpec((1, EMB), lambda i: (0, 0)),
    ],
    out_specs=pl.BlockSpec((NQ, _BM, QP), lambda i: (0, i, 0)),
    out_shape=jax.ShapeDtypeStruct((NQ, N, QP), jnp.float32),
)

_mlp_head = pl.pallas_call(
    _mlp_body,
    grid=(N // _BM,),
    in_specs=[
        pl.BlockSpec((NQ, _BM, QP), lambda i: (0, i, 0)),
        pl.BlockSpec((EMB, OUT), lambda i: (0, 0)),
        pl.BlockSpec((1, OUT), lambda i: (0, 0)),
    ],
    out_specs=pl.BlockSpec((_BM, OUT), lambda i: (i, 0)),
    out_shape=jax.ShapeDtypeStruct((N, OUT), jnp.float32),
)


def kernel(atom_feat, bond_feat, edge_index, W_atom, b_atom, W_bond, b_bond,
           Wg, bg, W_mlp, b_mlp):
    src = edge_index[0]
    dst = edge_index[1]
    src_pad = jnp.pad(src, (0, EPAD - E))
    dst_pad = jnp.pad(dst, (0, EPAD - E), constant_values=TRASH)
    src4 = (src_pad[None, :] +
            (N * jnp.arange(NQ, dtype=jnp.int32))[:, None]).reshape(-1)
    dst3 = dst_pad.reshape(NS, NCHUNK, CH)
    zeros = jnp.zeros((NPAD, QP), jnp.float32)

    hs = _embed_atoms(atom_feat, W_atom, b_atom.reshape(1, EMB))
    e = _embed_bonds(bond_feat, W_bond, b_bond.reshape(1, EMB))

    for i in range(LAYERS):
        aggr = _sc_layer(hs.reshape(NQ * N, QP), e, src4, dst3, zeros)
        hs = _layer_update(hs, aggr, Wg[i], bg[i].reshape(1, EMB))

    return _mlp_head(hs, W_mlp, b_mlp.reshape(1, OUT))
